# baseline (speedup 1.0000x reference)
import functools

import jax
import jax.numpy as jnp
from jax import lax
from jax.experimental import pallas as pl
from jax.experimental.pallas import tpu as pltpu
from jax.experimental.pallas import tpu_sc as plsc


def _sc_gather_body(idx_hbm, kflat_hbm, vflat_hbm, kout_hbm, vout_hbm,
                    idx_v, krows, vrows, sem):
    wid = lax.axis_index("s") * 2 + lax.axis_index("c")

    @pl.when(wid == 0)
    def _():
        pltpu.sync_copy(idx_hbm, idx_v)
        pltpu.async_copy(kflat_hbm.at[idx_v], krows, sem).wait()
        pltpu.async_copy(vflat_hbm.at[idx_v], vrows, sem).wait()
        pltpu.sync_copy(krows, kout_hbm)
        pltpu.sync_copy(vrows, vout_hbm)


def _matmul_kernel(rw_ref, krec_ref, vrec_ref, out_ref):
    rw = rw_ref[...]
    out_ref[0] = jnp.dot(rw, krec_ref[...], preferred_element_type=jnp.float32)
    out_ref[1] = jnp.dot(rw, vrec_ref[...], preferred_element_type=jnp.float32)


def kernel(keys, values, routing_weights, key_cache, value_cache,
           cache_positions):
    T, C, H = key_cache.shape
    B, S, _ = routing_weights.shape
    BS = B * S
    BS_BLK = 1024
    nblk = BS // BS_BLK

    rpos = ((cache_positions + 1) % C).astype(jnp.int32)
    flat_idx = jnp.arange(T, dtype=jnp.int32) * C + rpos
    kflat = key_cache.reshape(T * C, H)
    vflat = value_cache.reshape(T * C, H)

    mesh = plsc.VectorSubcoreMesh(core_axis_name="c", subcore_axis_name="s")
    sc_gather = functools.partial(
        pl.kernel,
        out_type=[jax.ShapeDtypeStruct((T, H), jnp.float32)] * 2,
        mesh=mesh,
        scratch_types=[
            pltpu.VMEM((T,), jnp.int32),
            pltpu.VMEM((T, H), jnp.float32),
            pltpu.VMEM((T, H), jnp.float32),
            pltpu.SemaphoreType.DMA,
        ],
    )(_sc_gather_body)
    recent_k, recent_v = sc_gather(flat_idx, kflat, vflat)

    rw2 = routing_weights.reshape(BS, T)
    out = pl.pallas_call(
        _matmul_kernel,
        grid=(nblk,),
        in_specs=[
            pl.BlockSpec((BS_BLK, T), lambda i: (i, 0)),
            pl.BlockSpec((T, H), lambda i: (0, 0)),
            pl.BlockSpec((T, H), lambda i: (0, 0)),
        ],
        out_specs=pl.BlockSpec((2, BS_BLK, H), lambda i: (0, i, 0)),
        out_shape=jax.ShapeDtypeStruct((2, BS, H), jnp.float32),
    )(rw2, recent_k, recent_v)
    return out.reshape(2, B, S, H)


# speedup vs baseline: 1.2784x; 1.2784x over previous
import jax
import jax.numpy as jnp
from jax.experimental import pallas as pl
from jax.experimental.pallas import tpu as pltpu


def _retrieve_kernel(rpos_ref, rw_ref, kcache_ref, vcache_ref, out_ref,
                     kscr, vscr, sem):
    i = pl.program_id(0)

    @pl.when(i == 0)
    def _gather():
        T = kscr.shape[0]
        copies = []
        for t in range(T):
            p = rpos_ref[t]
            copies.append(pltpu.make_async_copy(
                kcache_ref.at[t, pl.ds(p, 1), :], kscr.at[pl.ds(t, 1), :], sem))
            copies.append(pltpu.make_async_copy(
                vcache_ref.at[t, pl.ds(p, 1), :], vscr.at[pl.ds(t, 1), :], sem))
        for c in copies:
            c.start()
        for c in copies:
            c.wait()

    rw = rw_ref[...]
    out_ref[0] = jnp.dot(rw, kscr[...], preferred_element_type=jnp.float32)
    out_ref[1] = jnp.dot(rw, vscr[...], preferred_element_type=jnp.float32)


def kernel(keys, values, routing_weights, key_cache, value_cache,
           cache_positions):
    T, C, H = key_cache.shape
    B, S, _ = routing_weights.shape
    BS = B * S
    BS_BLK = 1024
    nblk = BS // BS_BLK

    rpos = ((cache_positions + 1) % C).astype(jnp.int32)
    rw2 = routing_weights.reshape(BS, T)

    grid_spec = pltpu.PrefetchScalarGridSpec(
        num_scalar_prefetch=1,
        grid=(nblk,),
        in_specs=[
            pl.BlockSpec((BS_BLK, T), lambda i, rpos_ref: (i, 0)),
            pl.BlockSpec(memory_space=pl.MemorySpace.ANY),
            pl.BlockSpec(memory_space=pl.MemorySpace.ANY),
        ],
        out_specs=pl.BlockSpec((2, BS_BLK, H), lambda i, rpos_ref: (0, i, 0)),
        scratch_shapes=[
            pltpu.VMEM((T, H), jnp.float32),
            pltpu.VMEM((T, H), jnp.float32),
            pltpu.SemaphoreType.DMA,
        ],
    )
    out = pl.pallas_call(
        _retrieve_kernel,
        grid_spec=grid_spec,
        out_shape=jax.ShapeDtypeStruct((2, BS, H), jnp.float32),
    )(rpos, rw2, key_cache, value_cache)
    return out.reshape(2, B, S, H)
